# baseline (speedup 1.0000x reference)
import jax
import jax.numpy as jnp
from jax.experimental import pallas as pl
from jax.experimental.pallas import tpu as pltpu

_IDX = tuple((63 * j) // 15 for j in range(16))
_FAST_CHUNKS = 4


def _dma_body(in_ref, slow_ref, fast_ref, fast_sem, slow_sem):
    nc, nt = in_ref.shape[0], in_ref.shape[1]
    tchunk = nt // _FAST_CHUNKS

    fast_dmas = []
    for c in range(nc):
        for i in range(_FAST_CHUNKS):
            d = pltpu.make_async_copy(
                in_ref.at[pl.ds(c, 1), pl.ds(i * tchunk, tchunk)],
                fast_ref.at[pl.ds(c, 1), pl.ds(i * tchunk, tchunk)],
                fast_sem,
            )
            d.start()
            fast_dmas.append(d)

    slow_dmas = []
    for j, src_t in enumerate(_IDX):
        d = pltpu.make_async_copy(
            in_ref.at[:, pl.ds(src_t, 1)],
            slow_ref.at[:, pl.ds(j, 1)],
            slow_sem,
        )
        d.start()
        slow_dmas.append(d)

    for d in slow_dmas:
        d.wait()
    for d in fast_dmas:
        d.wait()


def kernel(frames):
    c, t, h, w = frames.shape
    slow, fast = pl.pallas_call(
        _dma_body,
        in_specs=[pl.BlockSpec(memory_space=pl.ANY)],
        out_specs=[
            pl.BlockSpec(memory_space=pl.ANY),
            pl.BlockSpec(memory_space=pl.ANY),
        ],
        out_shape=[
            jax.ShapeDtypeStruct((c, t // 4, h, w), jnp.float32),
            jax.ShapeDtypeStruct((c, t, h, w), jnp.float32),
        ],
        scratch_shapes=[pltpu.SemaphoreType.DMA, pltpu.SemaphoreType.DMA],
    )(frames)
    return (slow, fast)


# speedup vs baseline: 47.5024x; 47.5024x over previous
import jax
import jax.numpy as jnp
from jax.experimental import pallas as pl
from jax.experimental.pallas import tpu as pltpu

_H = 256
_W = 256
_F = 8


def _pack_body(in_ref, slow_ref, fast_ref):
    fast_ref[...] = in_ref[...]
    k = pl.program_id(0)
    for s in range(_F // 4):
        jg = k * (_F // 4) + s
        j = jax.lax.rem(jg, 16)
        c = jax.lax.div(jg, 16)
        off = c * 64 + jax.lax.div(63 * j, 15) - k * _F
        slow_ref[pl.ds(s, 1)] = in_ref[pl.ds(off, 1)]


def _pack(frames_flat):
    n_blocks = frames_flat.shape[0] // _F
    return pl.pallas_call(
        _pack_body,
        grid=(n_blocks,),
        in_specs=[pl.BlockSpec((_F, _H, _W), lambda k: (k, 0, 0))],
        out_specs=[
            pl.BlockSpec((_F // 4, _H, _W), lambda k: (k, 0, 0)),
            pl.BlockSpec((_F, _H, _W), lambda k: (k, 0, 0)),
        ],
        out_shape=[
            jax.ShapeDtypeStruct((n_blocks * (_F // 4), _H, _W), jnp.float32),
            jax.ShapeDtypeStruct((n_blocks * _F, _H, _W), jnp.float32),
        ],
        compiler_params=pltpu.CompilerParams(
            dimension_semantics=("arbitrary",),
        ),
    )(frames_flat)


def kernel(frames):
    c, t, h, w = frames.shape
    flat = frames.reshape(c * t, h, w)
    slow, fast = _pack(flat)
    return (
        slow.reshape(c, t // 4, h, w),
        fast.reshape(c, t, h, w),
    )


# speedup vs baseline: 50.9002x; 1.0715x over previous
import jax
import jax.numpy as jnp
from jax.experimental import pallas as pl
from jax.experimental.pallas import tpu as pltpu

_H = 256
_W = 256
_F = 16


def _pack_body(in_ref, slow_ref, fast_ref):
    fast_ref[...] = in_ref[...]
    k = pl.program_id(0)
    for s in range(_F // 4):
        jg = k * (_F // 4) + s
        j = jax.lax.rem(jg, 16)
        c = jax.lax.div(jg, 16)
        off = c * 64 + jax.lax.div(63 * j, 15) - k * _F
        slow_ref[pl.ds(s, 1)] = in_ref[pl.ds(off, 1)]


def _pack(frames_flat):
    n_blocks = frames_flat.shape[0] // _F
    return pl.pallas_call(
        _pack_body,
        grid=(n_blocks,),
        in_specs=[pl.BlockSpec((_F, _H, _W), lambda k: (k, 0, 0))],
        out_specs=[
            pl.BlockSpec((_F // 4, _H, _W), lambda k: (k, 0, 0)),
            pl.BlockSpec((_F, _H, _W), lambda k: (k, 0, 0)),
        ],
        out_shape=[
            jax.ShapeDtypeStruct((n_blocks * (_F // 4), _H, _W), jnp.float32),
            jax.ShapeDtypeStruct((n_blocks * _F, _H, _W), jnp.float32),
        ],
        compiler_params=pltpu.CompilerParams(
            dimension_semantics=("arbitrary",),
        ),
    )(frames_flat)


def kernel(frames):
    c, t, h, w = frames.shape
    flat = frames.reshape(c * t, h, w)
    slow, fast = _pack(flat)
    return (
        slow.reshape(c, t // 4, h, w),
        fast.reshape(c, t, h, w),
    )


# speedup vs baseline: 54.0444x; 1.0618x over previous
import jax
import jax.numpy as jnp
from jax.experimental import pallas as pl
from jax.experimental.pallas import tpu as pltpu

_H = 256
_W = 256
_F = 32


def _pack_body(in_ref, slow_ref, fast_ref):
    fast_ref[...] = in_ref[...]
    k = pl.program_id(0)
    for s in range(_F // 4):
        jg = k * (_F // 4) + s
        j = jax.lax.rem(jg, 16)
        c = jax.lax.div(jg, 16)
        off = c * 64 + jax.lax.div(63 * j, 15) - k * _F
        slow_ref[pl.ds(s, 1)] = in_ref[pl.ds(off, 1)]


def _pack(frames_flat):
    n_blocks = frames_flat.shape[0] // _F
    return pl.pallas_call(
        _pack_body,
        grid=(n_blocks,),
        in_specs=[pl.BlockSpec((_F, _H, _W), lambda k: (k, 0, 0))],
        out_specs=[
            pl.BlockSpec((_F // 4, _H, _W), lambda k: (k, 0, 0)),
            pl.BlockSpec((_F, _H, _W), lambda k: (k, 0, 0)),
        ],
        out_shape=[
            jax.ShapeDtypeStruct((n_blocks * (_F // 4), _H, _W), jnp.float32),
            jax.ShapeDtypeStruct((n_blocks * _F, _H, _W), jnp.float32),
        ],
        compiler_params=pltpu.CompilerParams(
            dimension_semantics=("arbitrary",),
        ),
    )(frames_flat)


def kernel(frames):
    c, t, h, w = frames.shape
    flat = frames.reshape(c * t, h, w)
    slow, fast = _pack(flat)
    return (
        slow.reshape(c, t // 4, h, w),
        fast.reshape(c, t, h, w),
    )
